# Initial kernel scaffold; baseline (speedup 1.0000x reference)
import jax
import jax.numpy as jnp
from jax.experimental import pallas as pl


def kernel(x, token_table, pos_table):
    raise NotImplementedError("write your pallas kernel here")


# speedup vs baseline: 1.4811x; 1.4811x over previous
import jax
import jax.numpy as jnp
from jax import lax
from jax.experimental import pallas as pl
from jax.experimental.pallas import tpu as pltpu
from jax.experimental.pallas import tpu_sc as plsc

VOCAB = 1000000
MAXLEN = 200
EMBED_DIM = 32
BATCH = 4096

NC = 2
NS = 16
NW = NC * NS

TOTAL_ROWS = BATCH * MAXLEN
ROWS_PER_W = TOTAL_ROWS // NW
GATHER = 100
NGATHER = 16
CHUNK = GATHER * NGATHER
NCHUNK = ROWS_PER_W // CHUNK
REP = CHUNK // MAXLEN
NBUF = 2


def _body(x_hbm, tok_hbm, pos_hbm, out_hbm, idx_v, rows_v, pos_v,
          gsem, wsem):
    wid = lax.axis_index("s") * NC + lax.axis_index("c")
    w0 = wid * ROWS_PER_W

    pltpu.sync_copy(pos_hbm, pos_v)

    def load_and_fire(g, buf):
        base = pl.multiple_of(w0 + g * CHUNK, CHUNK)
        pltpu.sync_copy(
            x_hbm.at[pl.ds(pl.multiple_of(base // GATHER, NGATHER), NGATHER)],
            idx_v.at[buf],
        )
        for j in range(NGATHER):
            pltpu.async_copy(
                tok_hbm.at[idx_v.at[buf, j]],
                rows_v.at[buf, pl.ds(j * GATHER, GATHER)],
                gsem,
            )

    def drain_gathers(buf):
        for j in range(NGATHER):
            pltpu.make_async_copy(
                tok_hbm.at[idx_v.at[buf, j]],
                rows_v.at[buf, pl.ds(j * GATHER, GATHER)],
                gsem,
            ).wait()

    def add_pos(buf):
        def add_body(j, carry):
            p0 = pos_v[j, pl.ds(0, 16)]
            p1 = pos_v[j, pl.ds(16, 16)]
            for r in range(REP):
                row = r * MAXLEN + j
                rows_v[buf, row, pl.ds(0, 16)] += p0
                rows_v[buf, row, pl.ds(16, 16)] += p1
            return carry

        lax.fori_loop(0, MAXLEN, add_body, 0, unroll=2)

    def write_out(g, buf):
        base = pl.multiple_of(w0 + g * CHUNK, CHUNK)
        return pltpu.async_copy(rows_v.at[buf], out_hbm.at[pl.ds(base, CHUNK)],
                                wsem)

    def wait_write(g, buf):
        base = pl.multiple_of(w0 + g * CHUNK, CHUNK)
        pltpu.make_async_copy(rows_v.at[buf], out_hbm.at[pl.ds(base, CHUNK)],
                              wsem).wait()

    load_and_fire(0, 0)

    def outer(gg, carry):
        g0 = gg * NBUF
        for b in range(NBUF):
            g = g0 + b
            nxt_buf = (b + 1) % NBUF
            @pl.when(g > 0)
            def _():
                wait_write(g - 1, nxt_buf)

            @pl.when(g + 1 < NCHUNK)
            def _():
                load_and_fire(g + 1, nxt_buf)

            drain_gathers(b)
            add_pos(b)
            write_out(g, b)
        return carry

    lax.fori_loop(0, NCHUNK // NBUF, outer, 0)
    wait_write(NCHUNK - 1, (NCHUNK - 1) % NBUF)


@jax.jit
def _run(x_flat2d, token_table, pos_table):
    mesh = plsc.VectorSubcoreMesh(
        core_axis_name="c", subcore_axis_name="s",
        num_cores=NC, num_subcores=NS,
    )
    kern = pl.kernel(
        _body,
        out_type=jax.ShapeDtypeStruct((TOTAL_ROWS, EMBED_DIM), jnp.float32),
        mesh=mesh,
        scratch_types=[
            pltpu.VMEM((NBUF, NGATHER, GATHER), jnp.int32),
            pltpu.VMEM((NBUF, CHUNK, EMBED_DIM), jnp.float32),
            pltpu.VMEM((MAXLEN, EMBED_DIM), jnp.float32),
            pltpu.SemaphoreType.DMA,
            pltpu.SemaphoreType.DMA,
        ],
        compiler_params=pltpu.CompilerParams(use_tc_tiling_on_sc=False),
    )
    return kern(x_flat2d, token_table, pos_table)


def kernel(x, token_table, pos_table):
    x_flat2d = x.reshape(TOTAL_ROWS // GATHER, GATHER).astype(jnp.int32)
    out = _run(x_flat2d, token_table, pos_table)
    return out.reshape(BATCH, MAXLEN, EMBED_DIM)
